# baseline (speedup 1.0000x reference)
import functools

import jax
import jax.numpy as jnp
from jax import lax
from jax.experimental import pallas as pl
from jax.experimental.pallas import tpu as pltpu
from jax.experimental.pallas import tpu_sc as plsc


def _l1_body(x_ref, w_ref, b_ref, o_ref):
    acc = jnp.dot(x_ref[0], w_ref[...], preferred_element_type=jnp.float32)
    o_ref[0] = acc + b_ref[...]


def _conv_body(x_ref, w_ref, b_ref, o_ref, *, OH, OW, K):
    x = x_ref[0]
    lhs = jnp.concatenate([x[0:OH], x[1:OH + 1]], axis=2)
    y = jnp.dot(lhs.reshape(OH * OW, 2 * K), w_ref[...],
                preferred_element_type=jnp.float32)
    o_ref[0] = y + b_ref[...]


def _conv_layer(S, wk, bias, OH, OW, K, OC):
    N = S.shape[0]
    body = functools.partial(_conv_body, OH=OH, OW=OW, K=K)
    return pl.pallas_call(
        body,
        grid=(N,),
        in_specs=[
            pl.BlockSpec((1, OH + 1, OW, K), lambda n: (n, 0, 0, 0)),
            pl.BlockSpec((2 * K, OC), lambda n: (0, 0)),
            pl.BlockSpec((1, OC), lambda n: (0, 0)),
        ],
        out_specs=pl.BlockSpec((1, OH * OW, OC), lambda n: (n, 0, 0)),
        out_shape=jax.ShapeDtypeStruct((N, OH * OW, OC), jnp.float32),
    )(S, wk, bias)


def _s2d_pair(y):
    N, H, W, C = y.shape
    h2 = H // 2 + 1
    OW = h2 - 1
    xp = jnp.pad(y, ((0, 0), (1, 1), (1, 1), (0, 0)))
    r = xp.reshape(N, h2, 2, h2, 2, C).transpose(0, 1, 3, 2, 4, 5)
    r0 = r[:, :, :OW]
    r1 = r[:, :, 1:]
    s = jnp.stack([r0, r1], axis=4)
    return s.reshape(N, h2, OW, 8 * C)


def _wprep(w):
    OC, C = w.shape[0], w.shape[1]
    return w.astype(jnp.bfloat16).transpose(2, 3, 1, 0).reshape(16 * C, OC)


_BZ = 512
_BK = 2048


def _vq_body(zh_ref, zl_ref, cht_ref, clt_ref, z2_ref, c2_ref,
             idx_ref, dmin_ref):
    j = pl.program_id(1)
    t1 = jnp.dot(zl_ref[...], cht_ref[...], preferred_element_type=jnp.float32)
    t2 = jnp.dot(zh_ref[...], clt_ref[...], preferred_element_type=jnp.float32)
    t3 = jnp.dot(zh_ref[...], cht_ref[...], preferred_element_type=jnp.float32)
    zc = (t1 + t2) + t3
    d2 = (z2_ref[...] - 2.0 * zc) + c2_ref[...]
    dist = jnp.sqrt(jnp.maximum(d2, 0.0))
    m = jnp.min(dist, axis=1, keepdims=True)
    kio = lax.broadcasted_iota(jnp.int32, dist.shape, 1) + j * _BK
    li = jnp.min(jnp.where(dist == m, kio, jnp.int32(2 ** 30)),
                 axis=1, keepdims=True)

    @pl.when(j == 0)
    def _():
        dmin_ref[...] = m
        idx_ref[...] = li

    @pl.when(j > 0)
    def _():
        upd = m < dmin_ref[...]
        idx_ref[...] = jnp.where(upd, li, idx_ref[...])
        dmin_ref[...] = jnp.where(upd, m, dmin_ref[...])


def _vq_argmin(z_hi, z_lo, cht, clt, z2c, c2r):
    M, K = z_hi.shape[0], cht.shape[1]
    grid = (M // _BZ, K // _BK)
    return pl.pallas_call(
        _vq_body,
        grid=grid,
        in_specs=[
            pl.BlockSpec((_BZ, 256), lambda i, j: (i, 0)),
            pl.BlockSpec((_BZ, 256), lambda i, j: (i, 0)),
            pl.BlockSpec((256, _BK), lambda i, j: (0, j)),
            pl.BlockSpec((256, _BK), lambda i, j: (0, j)),
            pl.BlockSpec((_BZ, 1), lambda i, j: (i, 0)),
            pl.BlockSpec((1, _BK), lambda i, j: (0, j)),
        ],
        out_specs=[
            pl.BlockSpec((_BZ, 1), lambda i, j: (i, 0)),
            pl.BlockSpec((_BZ, 1), lambda i, j: (i, 0)),
        ],
        out_shape=[
            jax.ShapeDtypeStruct((M, 1), jnp.int32),
            jax.ShapeDtypeStruct((M, 1), jnp.float32),
        ],
    )(z_hi, z_lo, cht, clt, z2c, c2r)


def _sc_gather(table, idx):
    B, D = idx.shape[0], table.shape[1]
    info = plsc.get_sparse_core_info()
    NC, NS = info.num_cores, info.num_subcores
    NW = NC * NS
    bpw = B // NW
    mesh = plsc.VectorSubcoreMesh(core_axis_name="c", subcore_axis_name="s")

    @functools.partial(
        pl.kernel, mesh=mesh,
        out_type=jax.ShapeDtypeStruct((B, D), jnp.float32),
        scratch_types=[
            pltpu.VMEM((bpw,), jnp.int32),
            pltpu.VMEM((bpw, D), jnp.float32),
            pltpu.SemaphoreType.DMA,
        ],
    )
    def k(table_hbm, idx_hbm, out_hbm, idx_v, rows_v, sem):
        wid = lax.axis_index("s") * NC + lax.axis_index("c")
        base = wid * bpw
        pltpu.sync_copy(idx_hbm.at[pl.ds(base, bpw)], idx_v)
        pltpu.async_copy(table_hbm.at[idx_v], rows_v, sem).wait()
        pltpu.sync_copy(rows_v, out_hbm.at[pl.ds(base, bpw)])

    return k(table, idx)


def _finish(N, z_flat, codebook):
    z_hi = z_flat.astype(jnp.bfloat16)
    z_lo = (z_flat - z_hi.astype(jnp.float32)).astype(jnp.bfloat16)
    c_hi = codebook.astype(jnp.bfloat16)
    c_lo = (codebook - c_hi.astype(jnp.float32)).astype(jnp.bfloat16)
    z2c = jnp.sum(z_flat ** 2, axis=1, keepdims=True)
    c2r = jnp.sum(codebook ** 2, axis=1)[None, :]
    idx2d, dmin = _vq_argmin(z_hi, z_lo, c_hi.T, c_lo.T, z2c, c2r)
    idx = idx2d.reshape(-1)
    q = _sc_gather(codebook, idx)
    quantized_st = z_flat + (q - z_flat)
    quantized_st = quantized_st.reshape(N, 256, 256).transpose(0, 2, 1)
    quantized_st = quantized_st.reshape(N, 256, 16, 16)
    indices2 = idx.reshape(N, 256)
    commit_loss = jnp.sum(dmin * dmin) / jnp.float32(N * 256 * 256)
    return (quantized_st, indices2, commit_loss)


def kernel(images, w1, b1, w2, b2, w3, b3, w4, b4, codebook):
    N = images.shape[0]

    xb = images.astype(jnp.bfloat16).transpose(0, 2, 3, 1)
    xp = jnp.pad(xb, ((0, 0), (1, 1), (1, 1), (0, 0)))
    pats = [xp[:, kh:kh + 256:2, kw:kw + 256:2, :]
            for kh in range(4) for kw in range(4)]
    P1 = jnp.concatenate(pats, axis=-1).reshape(N, 128 * 128, 48)
    W1 = w1.astype(jnp.bfloat16).transpose(2, 3, 1, 0).reshape(48, 128)
    y1 = pl.pallas_call(
        _l1_body,
        grid=(N,),
        in_specs=[
            pl.BlockSpec((1, 128 * 128, 48), lambda n: (n, 0, 0)),
            pl.BlockSpec((48, 128), lambda n: (0, 0)),
            pl.BlockSpec((1, 128), lambda n: (0, 0)),
        ],
        out_specs=pl.BlockSpec((1, 128 * 128, 128), lambda n: (n, 0, 0)),
        out_shape=jax.ShapeDtypeStruct((N, 128 * 128, 128), jnp.float32),
    )(P1, W1, b1.reshape(1, -1))

    a1 = jax.nn.silu(y1).astype(jnp.bfloat16).reshape(N, 128, 128, 128)
    y2 = _conv_layer(_s2d_pair(a1), _wprep(w2), b2.reshape(1, -1),
                     64, 64, 1024, 256)
    a2 = jax.nn.silu(y2).astype(jnp.bfloat16).reshape(N, 64, 64, 256)
    y3 = _conv_layer(_s2d_pair(a2), _wprep(w3), b3.reshape(1, -1),
                     32, 32, 2048, 256)
    a3 = jax.nn.silu(y3).astype(jnp.bfloat16).reshape(N, 32, 32, 256)
    y4 = _conv_layer(_s2d_pair(a3), _wprep(w4), b4.reshape(1, -1),
                     16, 16, 2048, 256)
    z_flat = jax.nn.silu(y4).reshape(N * 256, 256)
    return _finish(N, z_flat, codebook)
